# baseline (speedup 1.0000x reference)
import functools

import jax
import jax.numpy as jnp
from jax import lax
from jax.experimental import pallas as pl
from jax.experimental.pallas import tpu as pltpu
from jax.experimental.pallas import tpu_sc as plsc

B = 16384
D = 128
NC = 2
NS = 16
L = 16
NW = NC * NS
BPW = B // NW
CH = 32
NCHUNK = BPW // CH
NG = CH // L

_mesh = plsc.VectorSubcoreMesh(core_axis_name="c", subcore_axis_name="s")


@functools.partial(
    pl.kernel,
    mesh=_mesh,
    compiler_params=pltpu.CompilerParams(needs_layout_passes=False),
    out_type=jax.ShapeDtypeStruct((B,), jnp.float32),
    scratch_types=[
        pltpu.VMEM((BPW,), jnp.int32),
        pltpu.VMEM((BPW,), jnp.int32),
        pltpu.VMEM((2, CH, D), jnp.float32),
        pltpu.VMEM((2, CH, D), jnp.float32),
        pltpu.VMEM((L * 17,), jnp.float32),
        pltpu.VMEM((BPW,), jnp.float32),
        pltpu.SemaphoreType.DMA((2,)),
        pltpu.SemaphoreType.DMA,
    ],
)
def _mf_sc(user_hbm, item_hbm, uemb_hbm, iemb_hbm, out_hbm,
           uidx_v, iidx_v, urows_v, irows_v, pbuf, outv, sems, semi):
    wid = lax.axis_index("s") * NC + lax.axis_index("c")
    base = wid * BPW

    ci = pltpu.async_copy(user_hbm.at[pl.ds(base, BPW)], uidx_v, semi)
    cj = pltpu.async_copy(item_hbm.at[pl.ds(base, BPW)], iidx_v, semi)
    ci.wait()
    cj.wait()

    lane17 = lax.iota(jnp.int32, L) * 17

    def start(c, b):
        pltpu.async_copy(
            uemb_hbm.at[uidx_v.at[pl.ds(c * CH, CH)]], urows_v.at[b], sems.at[b])
        pltpu.async_copy(
            iemb_hbm.at[iidx_v.at[pl.ds(c * CH, CH)]], irows_v.at[b], sems.at[b])

    start(0, 0)
    start(1, 1)

    def chunk_body(c, carry):
        b = lax.rem(c, 2)
        pltpu.make_async_copy(
            uemb_hbm.at[uidx_v.at[pl.ds(c * CH, CH)]], urows_v.at[b], sems.at[b]).wait()
        pltpu.make_async_copy(
            iemb_hbm.at[iidx_v.at[pl.ds(c * CH, CH)]], irows_v.at[b], sems.at[b]).wait()

        def group_body(g, carry2):
            def row_body(kk, carry3):
                for t in range(2):
                    k = 2 * kk + t
                    r = g * L + k
                    part = jnp.zeros((L,), jnp.float32)
                    for j in range(D // L):
                        uu = urows_v[b, r, pl.ds(j * L, L)]
                        vv = irows_v[b, r, pl.ds(j * L, L)]
                        part = part + uu * vv
                    plsc.store_scatter(pbuf, [lane17 + k], part)
                return carry3

            lax.fori_loop(0, L // 2, row_body, 0)
            acc = jnp.zeros((L,), jnp.float32)
            for j in range(L):
                acc = acc + pbuf[pl.ds(j * 17, L)]
            outv[pl.ds(c * CH + g * L, L)] = acc
            return carry2

        lax.fori_loop(0, NG, group_body, 0)

        @pl.when(c + 2 < NCHUNK)
        def _():
            start(c + 2, b)

        return carry

    lax.fori_loop(0, NCHUNK, chunk_body, 0)

    pltpu.sync_copy(outv, out_hbm.at[pl.ds(base, BPW)])


def kernel(user, item, users_emb, items_emb):
    return _mf_sc(user, item, users_emb, items_emb)


# speedup vs baseline: 1.0625x; 1.0625x over previous
import functools

import jax
import jax.numpy as jnp
from jax import lax
from jax.experimental import pallas as pl
from jax.experimental.pallas import tpu as pltpu
from jax.experimental.pallas import tpu_sc as plsc

B = 16384
D = 128
NC = 2
NS = 16
L = 16
NW = NC * NS
BPW = B // NW
CH = 64
NCHUNK = BPW // CH
NG = CH // L

_mesh = plsc.VectorSubcoreMesh(core_axis_name="c", subcore_axis_name="s")


@functools.partial(
    pl.kernel,
    mesh=_mesh,
    compiler_params=pltpu.CompilerParams(needs_layout_passes=False),
    out_type=jax.ShapeDtypeStruct((B,), jnp.float32),
    scratch_types=[
        pltpu.VMEM((BPW,), jnp.int32),
        pltpu.VMEM((BPW,), jnp.int32),
        pltpu.VMEM((3, CH, D), jnp.float32),
        pltpu.VMEM((3, CH, D), jnp.float32),
        pltpu.VMEM((L * 17,), jnp.float32),
        pltpu.VMEM((BPW,), jnp.float32),
        pltpu.SemaphoreType.DMA((3,)),
        pltpu.SemaphoreType.DMA,
    ],
)
def _mf_sc(user_hbm, item_hbm, uemb_hbm, iemb_hbm, out_hbm,
           uidx_v, iidx_v, urows_v, irows_v, pbuf, outv, sems, semi):
    wid = lax.axis_index("s") * NC + lax.axis_index("c")
    base = wid * BPW

    ci = pltpu.async_copy(user_hbm.at[pl.ds(base, BPW)], uidx_v, semi)
    cj = pltpu.async_copy(item_hbm.at[pl.ds(base, BPW)], iidx_v, semi)
    ci.wait()
    cj.wait()

    lane17 = lax.iota(jnp.int32, L) * 17

    def start(c, b):
        pltpu.async_copy(
            uemb_hbm.at[uidx_v.at[pl.ds(c * CH, CH)]], urows_v.at[b], sems.at[b])
        pltpu.async_copy(
            iemb_hbm.at[iidx_v.at[pl.ds(c * CH, CH)]], irows_v.at[b], sems.at[b])

    start(0, 0)
    start(1, 1)
    start(2, 2)

    def chunk_body(c, carry):
        b = lax.rem(c, 3)
        pltpu.make_async_copy(
            uemb_hbm.at[uidx_v.at[pl.ds(c * CH, CH)]], urows_v.at[b], sems.at[b]).wait()
        pltpu.make_async_copy(
            iemb_hbm.at[iidx_v.at[pl.ds(c * CH, CH)]], irows_v.at[b], sems.at[b]).wait()

        def group_body(g, carry2):
            def row_body(kk, carry3):
                for t in range(2):
                    k = 2 * kk + t
                    r = g * L + k
                    part = jnp.zeros((L,), jnp.float32)
                    for j in range(D // L):
                        uu = urows_v[b, r, pl.ds(j * L, L)]
                        vv = irows_v[b, r, pl.ds(j * L, L)]
                        part = part + uu * vv
                    plsc.store_scatter(pbuf, [lane17 + k], part)
                return carry3

            lax.fori_loop(0, L // 2, row_body, 0)
            acc = jnp.zeros((L,), jnp.float32)
            for j in range(L):
                acc = acc + pbuf[pl.ds(j * 17, L)]
            outv[pl.ds(c * CH + g * L, L)] = acc
            return carry2

        lax.fori_loop(0, NG, group_body, 0)

        @pl.when(c + 3 < NCHUNK)
        def _():
            start(c + 3, b)

        return carry

    lax.fori_loop(0, NCHUNK, chunk_body, 0)

    pltpu.sync_copy(outv, out_hbm.at[pl.ds(base, BPW)])


def kernel(user, item, users_emb, items_emb):
    return _mf_sc(user, item, users_emb, items_emb)
